# baseline (speedup 1.0000x reference)
import functools

import jax
import jax.numpy as jnp
from jax import lax
from jax.experimental import pallas as pl
from jax.experimental.pallas import tpu as pltpu
from jax.experimental.pallas import tpu_sc as plsc

_VOCAB = 1000000
_D = 64
_BATCH = 16384
_SEQ = 50
_N = _BATCH * _SEQ

_NC = 2
_NS = 16
_NW = _NC * _NS

_CHUNK = 128
_NUNIT = _N // _CHUNK
_PER_W = _NUNIT // _NW
_NBB = _BATCH // _CHUNK

_P = 6
_L = 16
_M = _D // _L
_TPAD = _CHUNK + 1


def _emb_body(idx_hbm, table_hbm, out_hbm, idx_v, g_v, t_v, gsems, ssems):
    wid = lax.axis_index("s") * _NC + lax.axis_index("c")
    base_u = wid * _PER_W
    pltpu.sync_copy(idx_hbm.at[wid], idx_v)

    def fire_gather(t, slot):
        pltpu.async_copy(table_hbm.at[idx_v.at[t]],
                         g_v.at[pl.ds(slot * _CHUNK, _CHUNK)],
                         gsems.at[slot])

    for p in range(_P):
        fire_gather(p, p)

    iota = lax.iota(jnp.int32, _L)
    dblk = [iota // 8 + 2 * m for m in range(_M)]
    drow = lax.rem(iota, 8)

    def transpose_chunk(slot, ts):
        dblk_ts = [v + ts * (_D // 8) for v in dblk]
        row0 = slot * _CHUNK
        vals = [g_v[row0, pl.ds(m * _L, _L)] for m in range(_M)]
        for b in range(_CHUNK):
            b_vec = jnp.full((_L,), b, jnp.int32)
            nxt = None
            if b + 1 < _CHUNK:
                nxt = [g_v[row0 + b + 1, pl.ds(m * _L, _L)]
                       for m in range(_M)]
            for m in range(_M):
                plsc.store_scatter(t_v, [dblk_ts[m], drow, b_vec], vals[m])
            vals = nxt

    def body(t, _):
        slot = lax.rem(t, _P)
        ts = lax.rem(t, 2)
        pltpu.make_async_copy(table_hbm.at[pl.ds(0, _CHUNK)],
                              g_v.at[pl.ds(slot * _CHUNK, _CHUNK)],
                              gsems.at[slot]).wait()
        @pl.when(t >= 2)
        def _():
            pltpu.make_async_copy(
                t_v.at[pl.ds(ts * (_D // 8), _D // 8), :, pl.ds(0, _CHUNK)],
                out_hbm.at[0, :, 0], ssems.at[ts]).wait()
        transpose_chunk(slot, ts)
        u = base_u + t
        s = u // _NBB
        jb = lax.rem(u, _NBB)
        pltpu.async_copy(
            t_v.at[pl.ds(ts * (_D // 8), _D // 8), :, pl.ds(0, _CHUNK)],
            out_hbm.at[s, :, jb], ssems.at[ts])
        @pl.when(t + _P < _PER_W)
        def _():
            fire_gather(t + _P, slot)
        return 0

    lax.fori_loop(0, _PER_W, body, 0)
    pltpu.make_async_copy(t_v.at[pl.ds(0, _D // 8), :, pl.ds(0, _CHUNK)],
                          out_hbm.at[0, :, 0], ssems.at[0]).wait()
    pltpu.make_async_copy(t_v.at[pl.ds(_D // 8, _D // 8), :, pl.ds(0, _CHUNK)],
                          out_hbm.at[0, :, 0], ssems.at[1]).wait()


_mesh = plsc.VectorSubcoreMesh(
    core_axis_name="c", subcore_axis_name="s",
    num_cores=_NC, num_subcores=_NS)

_emb = functools.partial(
    pl.kernel,
    out_type=jax.ShapeDtypeStruct((_SEQ, _D // 8, _NBB, 8, _CHUNK),
                                  jnp.float32),
    mesh=_mesh,
    scratch_types=[
        pltpu.VMEM((_PER_W, _CHUNK), jnp.int32),
        pltpu.VMEM((_P * _CHUNK, _D), jnp.float32),
        pltpu.VMEM((2 * (_D // 8), 8, _TPAD), jnp.float32),
        pltpu.SemaphoreType.DMA((_P,)),
        pltpu.SemaphoreType.DMA((2,)),
    ],
    compiler_params=pltpu.CompilerParams(use_tc_tiling_on_sc=False,
                                         needs_layout_passes=False),
)(_emb_body)


@jax.jit
def kernel(x, embed_weight):
    idx = x.T.reshape(_NW, _PER_W, _CHUNK).astype(jnp.int32)
    o5 = _emb(idx, embed_weight)
    out = o5.transpose(0, 1, 3, 2, 4).reshape(_SEQ, _D, _BATCH)
    return out.transpose(2, 0, 1)
